# baseline (speedup 1.0000x reference)
import jax
import jax.numpy as jnp
from jax.experimental import pallas as pl
from jax.experimental.pallas import tpu as pltpu
from jax.experimental.pallas import tpu_sc as plsc


def kernel(x, pos_table):
    B, S, N = x.shape
    x2 = x.reshape(B * S, N)
    pos = pos_table[:S]

    BR, BC = 64, 256
    LANES = 16

    mesh = plsc.VectorSubcoreMesh(core_axis_name="core",
                                  subcore_axis_name="subcore")

    @jax.jit
    def run(x2, pos):
        @pl.kernel(out_type=jax.ShapeDtypeStruct((B * S, N), x2.dtype),
                   mesh=mesh, scratch_types=[])
        def sc_kernel(x_hbm, p_hbm, o_hbm):
            def body(x_v, p_v, o_v):
                @pl.loop(0, BR)
                def _(r):
                    @pl.loop(0, BC, step=LANES)
                    def _(c):
                        slc = (pl.ds(r, 1), pl.ds(c, LANES))
                        o_v.at[*slc][...] = x_v.at[*slc][...] + p_v.at[*slc][...]

            pltpu.emit_pipeline(
                body,
                grid=(B * S // BR, N // BC),
                in_specs=[
                    pl.BlockSpec((BR, BC), index_map=lambda i, j: (i, j)),
                    pl.BlockSpec((BR, BC),
                                 index_map=lambda i, j: (i % (S // BR), j)),
                ],
                out_specs=[pl.BlockSpec((BR, BC),
                                        index_map=lambda i, j: (i, j))],
                core_axis_name=("core", "subcore"),
                dimension_semantics=(pltpu.PARALLEL, pltpu.PARALLEL),
            )(x_hbm, p_hbm, o_hbm)

        return sc_kernel(x2, pos)

    return run(x2, pos).reshape(B, S, N)


# speedup vs baseline: 1.3454x; 1.3454x over previous
import jax
import jax.numpy as jnp
from jax.experimental import pallas as pl
from jax.experimental.pallas import tpu as pltpu
from jax.experimental.pallas import tpu_sc as plsc


def _add_block(x_ref, pos_ref, o_ref):
    o_ref[...] = x_ref[...] + pos_ref[...]


def _tc_add(x, pos3):
    B, S, N = x.shape
    BS = 2048
    grid = (S // BS, B)
    return pl.pallas_call(
        _add_block,
        grid=grid,
        in_specs=[
            pl.BlockSpec((1, BS, N), lambda s, b: (b, s, 0)),
            pl.BlockSpec((1, BS, N), lambda s, b: (0, s, 0)),
        ],
        out_specs=pl.BlockSpec((1, BS, N), lambda s, b: (b, s, 0)),
        out_shape=jax.ShapeDtypeStruct((B, S, N), x.dtype),
    )(x, pos3)


def _sc_add(x2, pos):
    R, N = x2.shape
    S = pos.shape[0]
    BR, BC = 64, 256
    LANES = 16
    mesh = plsc.VectorSubcoreMesh(core_axis_name="core",
                                  subcore_axis_name="subcore")

    @pl.kernel(out_type=jax.ShapeDtypeStruct((R, N), x2.dtype),
               mesh=mesh, scratch_types=[])
    def sc_kernel(x_hbm, p_hbm, o_hbm):
        def body(x_v, p_v, o_v):
            @pl.loop(0, BR)
            def _(r):
                @pl.loop(0, BC, step=LANES)
                def _(c):
                    slc = (pl.ds(r, 1), pl.ds(c, LANES))
                    o_v.at[*slc][...] = x_v.at[*slc][...] + p_v.at[*slc][...]

        pltpu.emit_pipeline(
            body,
            grid=(R // BR, N // BC),
            in_specs=[
                pl.BlockSpec((BR, BC), index_map=lambda i, j: (i, j)),
                pl.BlockSpec((BR, BC),
                             index_map=lambda i, j: (i % (S // BR), j)),
            ],
            out_specs=[pl.BlockSpec((BR, BC), index_map=lambda i, j: (i, j))],
            core_axis_name=("core", "subcore"),
            dimension_semantics=(pltpu.PARALLEL, pltpu.PARALLEL),
        )(x_hbm, p_hbm, o_hbm)

    return sc_kernel(x2, pos)


def kernel(x, pos_table):
    B, S, N = x.shape
    pos = pos_table[:S]
    tc_out = _tc_add(x[: B - 1], pos[None])
    sc_out = _sc_add(x[B - 1], pos)
    return jnp.concatenate([tc_out, sc_out[None]], axis=0)


# speedup vs baseline: 3.8500x; 2.8617x over previous
import jax
import jax.numpy as jnp
from jax.experimental import pallas as pl
from jax.experimental.pallas import tpu as pltpu


def _make_add(BS):
    def _add_block(x_ref, pos_ref, o_ref):
        i = pl.program_id(0)
        o_ref[...] = x_ref[...] + pos_ref[pl.ds(i * BS, BS), :][None]
    return _add_block


def kernel(x, pos_table):
    B, S, N = x.shape
    BS = 2048
    grid = (S // BS, B)
    return pl.pallas_call(
        _make_add(BS),
        grid=grid,
        in_specs=[
            pl.BlockSpec((1, BS, N), lambda s, b: (b, s, 0)),
            pl.BlockSpec(memory_space=pltpu.VMEM),
        ],
        out_specs=pl.BlockSpec((1, BS, N), lambda s, b: (b, s, 0)),
        out_shape=jax.ShapeDtypeStruct((B, S, N), x.dtype),
    )(x, pos_table[:S])


# speedup vs baseline: 4.8971x; 1.2720x over previous
import jax
import jax.numpy as jnp
from jax.experimental import pallas as pl


def _add_block(x_ref, pos_ref, o_ref):
    o_ref[...] = x_ref[...] + pos_ref[...][None]


def kernel(x, pos_table):
    B, S, N = x.shape
    BS = 2048
    grid = (S // BS, B)
    return pl.pallas_call(
        _add_block,
        grid=grid,
        in_specs=[
            pl.BlockSpec((1, BS, N), lambda s, b: (b, s, 0)),
            pl.BlockSpec((BS, N), lambda s, b: (s, 0)),
        ],
        out_specs=pl.BlockSpec((1, BS, N), lambda s, b: (b, s, 0)),
        out_shape=jax.ShapeDtypeStruct((B, S, N), x.dtype),
    )(x, pos_table)
